# Initial kernel scaffold; baseline (speedup 1.0000x reference)
import jax
import jax.numpy as jnp
from jax.experimental import pallas as pl


def kernel(x, edge_index, W1, b1, W2, b2):
    raise NotImplementedError("write your pallas kernel here")


# speedup vs baseline: 10.1016x; 10.1016x over previous
import functools

import jax
import jax.numpy as jnp
from jax import lax
from jax.experimental import pallas as pl
from jax.experimental.pallas import tpu as pltpu
from jax.experimental.pallas import tpu_sc as plsc

NC, NS, LANES = 2, 16, 16
NW = NC * NS
CHUNK = 128
DEGW = 128
BM = 1000


def _mesh():
  return plsc.VectorSubcoreMesh(
      core_axis_name="c", subcore_axis_name="s", num_cores=NC, num_subcores=NS)


def _make_deg_kernel(npad, cpw, rpt):
  @functools.partial(
      pl.kernel,
      out_type=jax.ShapeDtypeStruct((NC, npad, DEGW), jnp.float32),
      mesh=_mesh(),
      scratch_types=[
          pltpu.VMEM((CHUNK,), jnp.int32),
          pltpu.VMEM((CHUNK, DEGW), jnp.float32),
          pltpu.VMEM_SHARED((npad, DEGW), jnp.float32),
          pltpu.SemaphoreType.DMA,
      ],
  )
  def deg_kernel(dst_hbm, ones_hbm, zeros_hbm, out_hbm, idx_v, ones_v, acc_sh, sem):
    c = lax.axis_index("c")
    s = lax.axis_index("s")
    wid = s * NC + c
    pltpu.sync_copy(zeros_hbm, acc_sh.at[pl.ds(s * rpt, rpt)])
    pltpu.sync_copy(ones_hbm, ones_v)
    plsc.subcore_barrier()

    def body(j, carry):
      pltpu.sync_copy(dst_hbm.at[wid * cpw + j], idx_v)
      pltpu.sync_copy(ones_v, acc_sh.at[idx_v], add=True)
      return carry

    lax.fori_loop(0, cpw, body, 0)
    plsc.subcore_barrier()
    pltpu.sync_copy(acc_sh.at[pl.ds(s * rpt, rpt)],
                    out_hbm.at[c, pl.ds(s * rpt, rpt)])

  return deg_kernel


def _make_edge_kernel(n, d, npad, cpw, rpt):
  @functools.partial(
      pl.kernel,
      out_type=jax.ShapeDtypeStruct((NC, npad, d), jnp.float32),
      mesh=_mesh(),
      scratch_types=[
          pltpu.VMEM((CHUNK,), jnp.int32),
          pltpu.VMEM((CHUNK,), jnp.int32),
          pltpu.VMEM((CHUNK, d), jnp.float32),
          pltpu.VMEM_SHARED((npad, d), jnp.float32),
          pltpu.SemaphoreType.DMA,
      ],
  )
  def edge_kernel(y_hbm, src_hbm, dst_hbm, zeros_hbm, out_hbm,
                  sidx, didx, rows_v, acc_sh, sem):
    c = lax.axis_index("c")
    s = lax.axis_index("s")
    wid = s * NC + c
    pltpu.sync_copy(zeros_hbm, acc_sh.at[pl.ds(s * rpt, rpt)])
    plsc.subcore_barrier()

    def body(j, carry):
      row = wid * cpw + j
      pltpu.sync_copy(src_hbm.at[row], sidx)
      pltpu.sync_copy(dst_hbm.at[row], didx)
      pltpu.async_copy(y_hbm.at[sidx], rows_v, sem).wait()
      pltpu.sync_copy(rows_v, acc_sh.at[didx], add=True)
      return carry

    lax.fori_loop(0, cpw, body, 0)
    plsc.subcore_barrier()
    pltpu.sync_copy(acc_sh.at[pl.ds(s * rpt, rpt)],
                    out_hbm.at[c, pl.ds(s * rpt, rpt)])

  return edge_kernel


def _dis_col(degp_ref):
  deg = degp_ref[0] + degp_ref[1] + 1.0
  return lax.rsqrt(deg[:, 0:1])


def _mm1_body(x_ref, w_ref, degp_ref, y_ref):
  xw = jnp.dot(x_ref[...], w_ref[...], preferred_element_type=jnp.float32)
  y_ref[...] = xw * _dis_col(degp_ref)


def _mm2_body(accp_ref, y_ref, degp_ref, b_ref, w_ref, y2_ref):
  dis = _dis_col(degp_ref)
  h = (accp_ref[0] + accp_ref[1] + y_ref[...]) * dis + b_ref[...]
  h = jnp.maximum(h, 0.0)
  y2_ref[...] = jnp.dot(h, w_ref[...], preferred_element_type=jnp.float32) * dis


def _mm3_body(accp_ref, y_ref, degp_ref, b_ref, out_ref):
  dis = _dis_col(degp_ref)
  out_ref[...] = (accp_ref[0] + accp_ref[1] + y_ref[...]) * dis + b_ref[...]


def kernel(x, edge_index, W1, b1, W2, b2):
  n, d = x.shape
  e = edge_index.shape[1]
  npad = -(-(n + 1) // (NS * 8)) * (NS * 8)
  rpt = npad // NS
  cpw = -(-e // (NW * CHUNK))
  epad = NW * cpw * CHUNK
  nchunks = NW * cpw

  src_p = jnp.concatenate(
      [edge_index[0], jnp.zeros((epad - e,), jnp.int32)]).reshape(nchunks, CHUNK)
  dst_p = jnp.concatenate(
      [edge_index[1], jnp.full((epad - e,), n, jnp.int32)]).reshape(nchunks, CHUNK)
  zeros_deg = jnp.zeros((rpt, DEGW), jnp.float32)
  ones_deg = jnp.ones((CHUNK, DEGW), jnp.float32)
  zeros_acc = jnp.zeros((rpt, d), jnp.float32)

  deg_kernel = _make_deg_kernel(npad, cpw, rpt)
  edge_kernel = _make_edge_kernel(n, d, npad, cpw, rpt)

  grid = n // BM
  w_spec = pl.BlockSpec((d, d), lambda i: (0, 0))
  row_spec = pl.BlockSpec((BM, d), lambda i: (i, 0))
  degp_spec = pl.BlockSpec((NC, BM, DEGW), lambda i: (0, i, 0))
  accp_spec = pl.BlockSpec((NC, BM, d), lambda i: (0, i, 0))
  b_spec = pl.BlockSpec((1, d), lambda i: (0, 0))

  mm1 = pl.pallas_call(
      _mm1_body,
      grid=(grid,),
      in_specs=[row_spec, w_spec, degp_spec],
      out_specs=row_spec,
      out_shape=jax.ShapeDtypeStruct((n, d), jnp.float32),
  )
  mm2 = pl.pallas_call(
      _mm2_body,
      grid=(grid,),
      in_specs=[accp_spec, row_spec, degp_spec, b_spec, w_spec],
      out_specs=row_spec,
      out_shape=jax.ShapeDtypeStruct((n, d), jnp.float32),
  )
  mm3 = pl.pallas_call(
      _mm3_body,
      grid=(grid,),
      in_specs=[accp_spec, row_spec, degp_spec, b_spec],
      out_specs=row_spec,
      out_shape=jax.ShapeDtypeStruct((n, d), jnp.float32),
  )

  degp = deg_kernel(dst_p, ones_deg, zeros_deg)
  y1 = mm1(x, W1, degp)
  accp1 = edge_kernel(y1, src_p, dst_p, zeros_acc)
  y2 = mm2(accp1, y1, degp, b1.reshape(1, d), W2)
  accp2 = edge_kernel(y2, src_p, dst_p, zeros_acc)
  return mm3(accp2, y2, degp, b2.reshape(1, d))
